# baseline (speedup 1.0000x reference)
import functools

import jax
import jax.numpy as jnp
from jax import lax
from jax.experimental import pallas as pl
from jax.experimental.pallas import tpu as pltpu
from jax.experimental.pallas import tpu_sc as plsc

_NUM_CLASSES = 100000
_BATCH = 1024
_TINY = 1e-7

_NC = 2
_NS = 16
_L = 16
_NW = _NC * _NS
_BPW = _BATCH // _NW
_LN2 = 0.6931471805599453


def _logf(x):
    bits = lax.bitcast_convert_type(x, jnp.int32)
    e = (bits >> 23) - 127
    m = lax.bitcast_convert_type((bits & 0x7FFFFF) | (127 << 23), jnp.float32)
    t = (m - 1.0) / (m + 1.0)
    u = t * t
    p = 2.0 * t * (1.0 + u * (1.0 / 3.0 + u * (1.0 / 5.0 + u * (1.0 / 7.0 + u * (1.0 / 9.0)))))
    return e.astype(jnp.float32) * _LN2 + p


def _make_sc_loss():
    mesh = plsc.VectorSubcoreMesh(
        core_axis_name="c", subcore_axis_name="s", num_cores=_NC)

    @functools.partial(
        pl.kernel,
        mesh=mesh,
        out_type=jax.ShapeDtypeStruct((_NW * _L,), jnp.float32),
        scratch_types=[
            pltpu.VMEM((_BPW,), jnp.int32),
            pltpu.VMEM((_BPW, 128), jnp.float32),
            pltpu.VMEM((_L,), jnp.float32),
            pltpu.SemaphoreType.DMA,
        ],
    )
    def k(y_hbm, logits_t_hbm, out_hbm, y_v, rows_v, acc_v, sem):
        wid = lax.axis_index("s") * _NC + lax.axis_index("c")
        base = pl.multiple_of(wid * _BPW, _BPW)
        colblk = (base // 128) * 128
        pltpu.sync_copy(y_hbm.at[pl.ds(base, _BPW)], y_v)
        lane = lax.iota(jnp.int32, _L)
        pltpu.async_copy(
            logits_t_hbm.at[y_v, pl.ds(colblk, 128)], rows_v, sem).wait()
        acc = jnp.zeros((_L,), jnp.float32)
        for c in range(_BPW // _L):
            s16 = (base - colblk) + c * _L
            res = jnp.zeros((_L,), jnp.float32)
            for j in range(_L):
                r = c * _L + j
                v16 = rows_v[r, pl.ds(s16, _L)]
                res = jnp.where(lane == j, v16[j], res)
            acc = acc + _logf(res + _TINY)
        acc_v[pl.ds(0, _L)] = acc
        pltpu.sync_copy(acc_v, out_hbm.at[pl.ds(wid * _L, _L)])

    return k


_sc_loss = _make_sc_loss()


def _tc_mean_body(x_ref, o_ref):
    o_ref[0, 0] = -jnp.sum(x_ref[...]) * (1.0 / _BATCH)


_tc_mean = pl.pallas_call(
    _tc_mean_body,
    out_shape=jax.ShapeDtypeStruct((1, 1), jnp.float32),
    out_specs=pl.BlockSpec(memory_space=pltpu.SMEM),
)


def kernel(y, logits):
    parts = _sc_loss(y, logits.T)
    return _tc_mean(parts.reshape(_NW * _L // 128, 128))[0, 0]


# speedup vs baseline: 1.0876x; 1.0876x over previous
import functools

import jax
import jax.numpy as jnp
from jax import lax
from jax.experimental import pallas as pl
from jax.experimental.pallas import tpu as pltpu
from jax.experimental.pallas import tpu_sc as plsc

_NUM_CLASSES = 100000
_BATCH = 1024
_TINY = 1e-7

_NC = 1
_NS = 16
_L = 16
_NW = _NC * _NS
_BPW = _BATCH // _NW
_LN2 = 0.6931471805599453


def _logf(x):
    bits = lax.bitcast_convert_type(x, jnp.int32)
    e = (bits >> 23) - 127
    m = lax.bitcast_convert_type((bits & 0x7FFFFF) | (127 << 23), jnp.float32)
    t = (m - 1.0) / (m + 1.0)
    u = t * t
    p = 2.0 * t * (1.0 + u * (1.0 / 3.0 + u * (1.0 / 5.0 + u * (1.0 / 7.0 + u * (1.0 / 9.0)))))
    return e.astype(jnp.float32) * _LN2 + p


def _make_sc_loss():
    mesh = plsc.VectorSubcoreMesh(
        core_axis_name="c", subcore_axis_name="s", num_cores=_NC)

    @functools.partial(
        pl.kernel,
        mesh=mesh,
        out_type=jax.ShapeDtypeStruct((_L,), jnp.float32),
        scratch_types=[
            pltpu.VMEM((_BPW,), jnp.int32),
            pltpu.VMEM((_BPW, 128), jnp.float32),
            pltpu.VMEM((_L,), jnp.float32),
            pltpu.VMEM((_NW * _L,), jnp.float32),
            pltpu.VMEM_SHARED((_NW * _L,), jnp.float32),
            pltpu.SemaphoreType.DMA,
        ],
    )
    def k(y_hbm, logits_t_hbm, out_hbm, y_v, rows_v, acc_v, all_v, shared, sem):
        wid = lax.axis_index("s")
        base = pl.multiple_of(wid * _BPW, _BPW)
        colblk = (base // 128) * 128
        pltpu.sync_copy(y_hbm.at[pl.ds(base, _BPW)], y_v)
        lane = lax.iota(jnp.int32, _L)
        pltpu.async_copy(
            logits_t_hbm.at[y_v, pl.ds(colblk, 128)], rows_v, sem).wait()
        acc = jnp.zeros((_L,), jnp.float32)
        for c in range(_BPW // _L):
            s16 = (base - colblk) + c * _L
            res = jnp.zeros((_L,), jnp.float32)
            for j in range(_L):
                r = c * _L + j
                v16 = rows_v[r, pl.ds(s16, _L)]
                res = jnp.where(lane == j, v16[j], res)
            acc = acc + _logf(res + _TINY)
        acc_v[pl.ds(0, _L)] = acc
        pltpu.sync_copy(acc_v, shared.at[pl.ds(wid * _L, _L)])
        plsc.subcore_barrier()

        @pl.when(wid == 0)
        def _():
            pltpu.sync_copy(shared, all_v)
            tot = jnp.zeros((_L,), jnp.float32)
            for w in range(_NW):
                tot = tot + all_v[pl.ds(w * _L, _L)]
            total = tot[0]
            for l in range(1, _L):
                total = total + tot[l]
            acc_v[pl.ds(0, _L)] = jnp.where(
                lane >= 0, total, 0.0) * (-1.0 / _BATCH)
            pltpu.sync_copy(acc_v, out_hbm)

    return k


_sc_loss = _make_sc_loss()


def kernel(y, logits):
    return _sc_loss(y, logits.T)[0]
